# baseline (speedup 1.0000x reference)
import functools

import jax
import jax.numpy as jnp
from jax import lax
from jax.experimental import pallas as pl
from jax.experimental.pallas import tpu as pltpu
from jax.experimental.pallas import tpu_sc as plsc

FACT_DIM = 128
D_MODEL = 1024

_NC = 2
_NS = 16
_NW = _NC * _NS

_IDX_CHUNK = 128

_N_CHUNKS = 1
_BLK = 4096


def _gather_body(table_hbm, idx_hbm, out_hbm, idx_v, rows_v, sem, b_per_w):
    wid = lax.axis_index("s") * _NC + lax.axis_index("c")
    base = wid * b_per_w
    pltpu.sync_copy(idx_hbm.at[pl.ds(base, b_per_w)], idx_v)
    n = b_per_w // _IDX_CHUNK
    copies = []
    for j in range(n):
        copies.append(
            pltpu.async_copy(
                table_hbm.at[idx_v.at[pl.ds(j * _IDX_CHUNK, _IDX_CHUNK)]],
                rows_v.at[pl.ds(j * _IDX_CHUNK, _IDX_CHUNK)],
                sem,
            )
        )
    for c in copies:
        c.wait()
    pltpu.sync_copy(rows_v, out_hbm.at[pl.ds(base, b_per_w)])


def _sc_gather(table, idx):
    b = idx.shape[0]
    b_per_w = b // _NW
    mesh = plsc.VectorSubcoreMesh(core_axis_name="c", subcore_axis_name="s")
    return pl.kernel(
        functools.partial(_gather_body, b_per_w=b_per_w),
        out_type=jax.ShapeDtypeStruct((b, FACT_DIM), jnp.float32),
        mesh=mesh,
        scratch_types=[
            pltpu.VMEM((b_per_w,), jnp.int32),
            pltpu.VMEM((b_per_w, FACT_DIM), jnp.float32),
            pltpu.SemaphoreType.DMA,
        ],
    )(table, idx)


def _matmul_first_body(x_ref, w_ref, o_ref):
    o_ref[...] = lax.dot_general(
        x_ref[...].astype(jnp.bfloat16),
        w_ref[...].astype(jnp.bfloat16),
        (((1,), (1,)), ((), ())),
        preferred_element_type=jnp.float32,
    )


def _matmul_chain_body(x_ref, w_ref, buf_ref, o_ref):
    del buf_ref
    o_ref[...] = lax.dot_general(
        x_ref[...].astype(jnp.bfloat16),
        w_ref[...].astype(jnp.bfloat16),
        (((1,), (1,)), ((), ())),
        preferred_element_type=jnp.float32,
    )


def _tc_project_chunk(rows, w, buf, total_rows, row_offset):
    n_blk = rows.shape[0] // _BLK
    blk_off = row_offset // _BLK
    out_shape = jax.ShapeDtypeStruct((total_rows, D_MODEL), jnp.float32)
    x_spec = pl.BlockSpec((_BLK, FACT_DIM), lambda i: (i, 0))
    w_spec = pl.BlockSpec((D_MODEL, FACT_DIM), lambda i: (0, 0))
    o_spec = pl.BlockSpec((_BLK, D_MODEL), lambda i, _o=blk_off: (i + _o, 0))
    if buf is None:
        return pl.pallas_call(
            _matmul_first_body,
            grid=(n_blk,),
            in_specs=[x_spec, w_spec],
            out_specs=o_spec,
            out_shape=out_shape,
        )(rows, w)
    return pl.pallas_call(
        _matmul_chain_body,
        grid=(n_blk,),
        in_specs=[x_spec, w_spec, pl.BlockSpec(memory_space=pl.ANY)],
        out_specs=o_spec,
        out_shape=out_shape,
        input_output_aliases={2: 0},
    )(rows, w, buf)


def kernel(input_ids, token_embedding, projection_weight):
    batch, seq = input_ids.shape
    total = batch * seq
    idx = input_ids.reshape(-1).astype(jnp.int32)
    chunk = total // _N_CHUNKS
    gathered = [
        _sc_gather(token_embedding, lax.slice(idx, (k * chunk,), ((k + 1) * chunk,)))
        for k in range(_N_CHUNKS)
    ]
    buf = None
    for k in range(_N_CHUNKS):
        buf = _tc_project_chunk(gathered[k], projection_weight, buf, total, k * chunk)
    return buf.reshape(batch, seq, D_MODEL)


# speedup vs baseline: 1.0248x; 1.0248x over previous
import functools

import jax
import jax.numpy as jnp
from jax import lax
from jax.experimental import pallas as pl
from jax.experimental.pallas import tpu as pltpu
from jax.experimental.pallas import tpu_sc as plsc

FACT_DIM = 128
D_MODEL = 1024

_NC = 2
_NS = 16
_NW = _NC * _NS

_IDX_CHUNK = 128

_N_CHUNKS = 1
_BLK = 2048


def _gather_body(table_hbm, idx_hbm, out_hbm, idx_v, rows_v, sem, b_per_w):
    wid = lax.axis_index("s") * _NC + lax.axis_index("c")
    base = wid * b_per_w
    seq = idx_hbm.shape[1]
    per_row = seq // b_per_w
    row = wid // per_row
    col0 = (wid % per_row) * b_per_w
    pltpu.sync_copy(idx_hbm.at[row, pl.ds(col0, b_per_w)], idx_v)
    n = b_per_w // _IDX_CHUNK
    copies = []
    for j in range(n):
        copies.append(
            pltpu.async_copy(
                table_hbm.at[idx_v.at[pl.ds(j * _IDX_CHUNK, _IDX_CHUNK)]],
                rows_v.at[pl.ds(j * _IDX_CHUNK, _IDX_CHUNK)],
                sem,
            )
        )
    for c in copies:
        c.wait()
    pltpu.sync_copy(rows_v, out_hbm.at[pl.ds(base, b_per_w)])


def _sc_gather(table, idx):
    b = idx.shape[0] * idx.shape[1]
    b_per_w = b // _NW
    mesh = plsc.VectorSubcoreMesh(core_axis_name="c", subcore_axis_name="s")
    return pl.kernel(
        functools.partial(_gather_body, b_per_w=b_per_w),
        out_type=jax.ShapeDtypeStruct((b, FACT_DIM), jnp.float32),
        mesh=mesh,
        scratch_types=[
            pltpu.VMEM((b_per_w,), jnp.int32),
            pltpu.VMEM((b_per_w, FACT_DIM), jnp.float32),
            pltpu.SemaphoreType.DMA,
        ],
    )(table, idx)


def _matmul_first_body(x_ref, w_ref, o_ref):
    o_ref[...] = lax.dot_general(
        x_ref[...].astype(jnp.bfloat16),
        w_ref[...].astype(jnp.bfloat16),
        (((1,), (1,)), ((), ())),
        preferred_element_type=jnp.float32,
    )


def _matmul_chain_body(x_ref, w_ref, buf_ref, o_ref):
    del buf_ref
    o_ref[...] = lax.dot_general(
        x_ref[...].astype(jnp.bfloat16),
        w_ref[...].astype(jnp.bfloat16),
        (((1,), (1,)), ((), ())),
        preferred_element_type=jnp.float32,
    )


def _tc_project_chunk(rows, w, buf, total_rows, row_offset):
    n_blk = rows.shape[0] // _BLK
    blk_off = row_offset // _BLK
    out_shape = jax.ShapeDtypeStruct((total_rows, D_MODEL), jnp.float32)
    x_spec = pl.BlockSpec((_BLK, FACT_DIM), lambda i: (i, 0))
    w_spec = pl.BlockSpec((D_MODEL, FACT_DIM), lambda i: (0, 0))
    o_spec = pl.BlockSpec((_BLK, D_MODEL), lambda i, _o=blk_off: (i + _o, 0))
    if buf is None:
        return pl.pallas_call(
            _matmul_first_body,
            grid=(n_blk,),
            in_specs=[x_spec, w_spec],
            out_specs=o_spec,
            out_shape=out_shape,
        )(rows, w)
    return pl.pallas_call(
        _matmul_chain_body,
        grid=(n_blk,),
        in_specs=[x_spec, w_spec, pl.BlockSpec(memory_space=pl.ANY)],
        out_specs=o_spec,
        out_shape=out_shape,
        input_output_aliases={2: 0},
    )(rows, w, buf)


def kernel(input_ids, token_embedding, projection_weight):
    batch, seq = input_ids.shape
    total = batch * seq
    rows = _sc_gather(token_embedding, input_ids)
    buf = _tc_project_chunk(rows, projection_weight, None, total, 0)
    return buf.reshape(batch, seq, D_MODEL)
